# Initial kernel scaffold; baseline (speedup 1.0000x reference)
import jax
import jax.numpy as jnp
from jax.experimental import pallas as pl


def kernel(daytime, weekday, day, daytime_table, weekday_table, day_table):
    raise NotImplementedError("write your pallas kernel here")


# speedup vs baseline: 1.6696x; 1.6696x over previous
import functools

import jax
import jax.numpy as jnp
from jax import lax
from jax.experimental import pallas as pl
from jax.experimental.pallas import tpu as pltpu
from jax.experimental.pallas import tpu_sc as plsc

B, L, D = 4096, 200, 128
N = B * L
NC, NS, LANES = 2, 16, 16
NW = NC * NS
TOK_PER_W = N // NW
K = 256
NCHUNK = TOK_PER_W // K


def _emb_body(dt_idx, wd_idx, dy_idx, dt_tab, wd_tab, dy_tab, out,
              i1, i2, i3, r1, r2, r3, sem):
    wid = lax.axis_index("s") * NC + lax.axis_index("c")
    w_base = wid * TOK_PER_W

    def chunk(c, _):
        base = w_base + c * K
        pltpu.sync_copy(dt_idx.at[pl.ds(base, K)], i1)
        pltpu.sync_copy(wd_idx.at[pl.ds(base, K)], i2)
        pltpu.sync_copy(dy_idx.at[pl.ds(base, K)], i3)
        g1 = pltpu.async_copy(dt_tab.at[i1], r1, sem)
        g2 = pltpu.async_copy(wd_tab.at[i2], r2, sem)
        g3 = pltpu.async_copy(dy_tab.at[i3], r3, sem)
        g1.wait()
        g2.wait()
        g3.wait()

        def row(i, _):
            for j in range(D // LANES):
                s = pl.ds(j * LANES, LANES)
                r1[i, s] = r1[i, s] + r2[i, s] + r3[i, s]
            return ()

        lax.fori_loop(0, K, row, (), unroll=4)
        pltpu.sync_copy(r1, out.at[pl.ds(base, K)])
        return ()

    lax.fori_loop(0, NCHUNK, chunk, ())


@functools.partial(jax.jit, static_argnames=())
def kernel(daytime, weekday, day, daytime_table, weekday_table, day_table):
    dt = daytime.reshape(N).astype(jnp.int32)
    wd = weekday.reshape(N).astype(jnp.int32)
    dy = day.reshape(N).astype(jnp.int32)

    mesh = plsc.VectorSubcoreMesh(core_axis_name="c", subcore_axis_name="s")
    run = pl.kernel(
        _emb_body,
        out_type=jax.ShapeDtypeStruct((N, D), jnp.float32),
        mesh=mesh,
        scratch_types=[
            pltpu.VMEM((K,), jnp.int32),
            pltpu.VMEM((K,), jnp.int32),
            pltpu.VMEM((K,), jnp.int32),
            pltpu.VMEM((K, D), jnp.float32),
            pltpu.VMEM((K, D), jnp.float32),
            pltpu.VMEM((K, D), jnp.float32),
            pltpu.SemaphoreType.DMA,
        ],
    )
    out = run(dt, wd, dy, daytime_table, weekday_table, day_table)
    return out.reshape(B, L, D)


# speedup vs baseline: 1.6996x; 1.0180x over previous
import functools

import jax
import jax.numpy as jnp
from jax import lax
from jax.experimental import pallas as pl
from jax.experimental.pallas import tpu as pltpu
from jax.experimental.pallas import tpu_sc as plsc

B, L, D = 4096, 200, 128
N = B * L
NC, NS, LANES = 2, 16, 16
NW = NC * NS
TOK_PER_W = N // NW
K = 256
NCHUNK = TOK_PER_W // K


def _emb_body(dt_idx, wd_idx, dy_idx, dt_tab, wd_tab, dy_tab, out,
              i1, i2, i3, r1, r2, r3, sem):
    wid = lax.axis_index("s") * NC + lax.axis_index("c")
    w_base = wid * TOK_PER_W

    def chunk(c, _):
        base = w_base + c * K
        pltpu.sync_copy(dt_idx.at[pl.ds(base, K)], i1)
        pltpu.sync_copy(wd_idx.at[pl.ds(base, K)], i2)
        pltpu.sync_copy(dy_idx.at[pl.ds(base, K)], i3)
        pltpu.async_copy(dt_tab.at[i1], r1, sem).wait()
        pltpu.async_copy(wd_tab.at[i2], r1, sem, add=True).wait()
        pltpu.async_copy(dy_tab.at[i3], r1, sem, add=True).wait()
        pltpu.sync_copy(r1, out.at[pl.ds(base, K)])
        return ()

    lax.fori_loop(0, NCHUNK, chunk, ())


@functools.partial(jax.jit, static_argnames=())
def kernel(daytime, weekday, day, daytime_table, weekday_table, day_table):
    dt = daytime.reshape(N).astype(jnp.int32)
    wd = weekday.reshape(N).astype(jnp.int32)
    dy = day.reshape(N).astype(jnp.int32)

    mesh = plsc.VectorSubcoreMesh(core_axis_name="c", subcore_axis_name="s")
    run = pl.kernel(
        _emb_body,
        out_type=jax.ShapeDtypeStruct((N, D), jnp.float32),
        mesh=mesh,
        scratch_types=[
            pltpu.VMEM((K,), jnp.int32),
            pltpu.VMEM((K,), jnp.int32),
            pltpu.VMEM((K,), jnp.int32),
            pltpu.VMEM((K, D), jnp.float32),
            pltpu.VMEM((K, D), jnp.float32),
            pltpu.VMEM((K, D), jnp.float32),
            pltpu.SemaphoreType.DMA,
        ],
    )
    out = run(dt, wd, dy, daytime_table, weekday_table, day_table)
    return out.reshape(B, L, D)


# speedup vs baseline: 15.2272x; 8.9591x over previous
import functools

import jax
import jax.numpy as jnp
from jax import lax
from jax.experimental import pallas as pl
from jax.experimental.pallas import tpu as pltpu
from jax.experimental.pallas import tpu_sc as plsc

B, L, D = 4096, 200, 128
N = B * L
NC, NS = 2, 16
NW = NC * NS
TOK_PER_W = N // NW
K = 256
NCHUNK = TOK_PER_W // K
V_DT, V_WD, V_DY = 1441, 8, 367


def _emb_body(dt_idx, wd_idx, dy_idx, dt_tab, wd_tab, dy_tab, out,
              dt_s, wd_s, dy_s, i1, i2, i3, r1, sem):
    sid = lax.axis_index("s")
    wid = sid * NC + lax.axis_index("c")
    w_base = wid * TOK_PER_W

    @pl.when(sid == 0)
    def _():
        pltpu.sync_copy(dt_tab, dt_s)

    @pl.when(sid == 1)
    def _():
        pltpu.sync_copy(wd_tab, wd_s)

    @pl.when(sid == 2)
    def _():
        pltpu.sync_copy(dy_tab, dy_s)

    pltpu.sync_copy(dt_idx.at[pl.ds(w_base, TOK_PER_W)], i1)
    pltpu.sync_copy(wd_idx.at[pl.ds(w_base, TOK_PER_W)], i2)
    pltpu.sync_copy(dy_idx.at[pl.ds(w_base, TOK_PER_W)], i3)
    plsc.subcore_barrier()

    def chunk(c, _):
        s = pl.ds(c * K, K)
        pltpu.async_copy(dt_s.at[i1.at[s]], r1, sem).wait()
        pltpu.async_copy(wd_s.at[i2.at[s]], r1, sem, add=True).wait()
        pltpu.async_copy(dy_s.at[i3.at[s]], r1, sem, add=True).wait()
        pltpu.sync_copy(r1, out.at[pl.ds(w_base + c * K, K)])
        return ()

    lax.fori_loop(0, NCHUNK, chunk, ())


@functools.partial(jax.jit, static_argnames=())
def kernel(daytime, weekday, day, daytime_table, weekday_table, day_table):
    dt = daytime.reshape(N).astype(jnp.int32)
    wd = weekday.reshape(N).astype(jnp.int32)
    dy = day.reshape(N).astype(jnp.int32)

    mesh = plsc.VectorSubcoreMesh(core_axis_name="c", subcore_axis_name="s")
    run = pl.kernel(
        _emb_body,
        out_type=jax.ShapeDtypeStruct((N, D), jnp.float32),
        mesh=mesh,
        scratch_types=[
            pltpu.VMEM_SHARED((V_DT, D), jnp.float32),
            pltpu.VMEM_SHARED((V_WD, D), jnp.float32),
            pltpu.VMEM_SHARED((V_DY, D), jnp.float32),
            pltpu.VMEM((TOK_PER_W,), jnp.int32),
            pltpu.VMEM((TOK_PER_W,), jnp.int32),
            pltpu.VMEM((TOK_PER_W,), jnp.int32),
            pltpu.VMEM((K, D), jnp.float32),
            pltpu.SemaphoreType.DMA,
        ],
    )
    out = run(dt, wd, dy, daytime_table, weekday_table, day_table)
    return out.reshape(B, L, D)


# speedup vs baseline: 23.9214x; 1.5710x over previous
import functools

import jax
import jax.numpy as jnp
from jax import lax
from jax.experimental import pallas as pl
from jax.experimental.pallas import tpu as pltpu
from jax.experimental.pallas import tpu_sc as plsc

B, L, D = 4096, 200, 128
N = B * L
NC, NS = 2, 16
NW = NC * NS
TOK_PER_W = N // NW
K = 256
NCHUNK = TOK_PER_W // K
V_DT, V_WD, V_DY = 1441, 8, 367
V_CB = 3072
CB_PER_TILE = V_CB // NS
LANES = 16


def _emb_body(dt_idx, wd_idx, dy_idx, dt_tab, wd_tab, dy_tab, out,
              dt_s, wd_s, dy_s, cb_s, i1, i2, i3, i23, ci_dy, ci_wd, r,
              sem_i, sem_g, sem_o):
    sid = lax.axis_index("s")
    wid = sid * NC + lax.axis_index("c")
    w_base = wid * TOK_PER_W

    @pl.when(sid == 0)
    def _():
        pltpu.sync_copy(dt_tab, dt_s)

    @pl.when(sid == 1)
    def _():
        pltpu.sync_copy(wd_tab, wd_s)

    @pl.when(sid == 2)
    def _():
        pltpu.sync_copy(dy_tab, dy_s)

    plsc.subcore_barrier()

    cb_base = sid * CB_PER_TILE
    for j in range(CB_PER_TILE // LANES):
        v = cb_base + j * LANES + lax.iota(jnp.int32, 16)
        ci_dy[pl.ds(j * LANES, LANES)] = jnp.minimum(v >> 3, V_DY - 1)
        ci_wd[pl.ds(j * LANES, LANES)] = v & 7
    cbuf = r.at[0, pl.ds(0, CB_PER_TILE)]
    pltpu.async_copy(dy_s.at[ci_dy], cbuf, sem_g).wait()
    pltpu.async_copy(wd_s.at[ci_wd], cbuf, sem_g, add=True).wait()
    pltpu.sync_copy(cbuf, cb_s.at[pl.ds(cb_base, CB_PER_TILE)])
    plsc.subcore_barrier()

    def prefetch_idx(c, b):
        s = pl.ds(w_base + c * K, K)
        d = pl.ds(b * K, K)
        pltpu.async_copy(dt_idx.at[s], i1.at[d], sem_i.at[b])
        pltpu.async_copy(wd_idx.at[s], i2.at[d], sem_i.at[b])
        pltpu.async_copy(dy_idx.at[s], i3.at[d], sem_i.at[b])

    def wait_idx(b):
        d = pl.ds(b * K, K)
        pltpu.make_async_copy(dt_idx.at[pl.ds(0, K)], i1.at[d], sem_i.at[b]).wait()
        pltpu.make_async_copy(wd_idx.at[pl.ds(0, K)], i2.at[d], sem_i.at[b]).wait()
        pltpu.make_async_copy(dy_idx.at[pl.ds(0, K)], i3.at[d], sem_i.at[b]).wait()

    prefetch_idx(0, 0)
    prefetch_idx(1, 1)

    def chunk(c, b):
        @pl.when(c >= 2)
        def _():
            pltpu.make_async_copy(r.at[b], out.at[pl.ds(w_base, K)],
                                  sem_o.at[b]).wait()

        wait_idx(b)
        d = pl.ds(b * K, K)
        for j in range(K // LANES):
            s16 = pl.ds(b * K + j * LANES, LANES)
            i23[s16] = (i3[s16] << 3) + i2[s16]
        pltpu.async_copy(dt_s.at[i1.at[d]], r.at[b], sem_g).wait()
        pltpu.async_copy(cb_s.at[i23.at[d]], r.at[b], sem_g, add=True).wait()
        pltpu.async_copy(r.at[b], out.at[pl.ds(w_base + c * K, K)], sem_o.at[b])

        @pl.when(c + 2 < NCHUNK)
        def _():
            prefetch_idx(c + 2, b)

    def pair(p, _):
        chunk(2 * p, 0)
        chunk(2 * p + 1, 1)
        return ()

    lax.fori_loop(0, NCHUNK // 2, pair, ())

    pltpu.make_async_copy(r.at[0], out.at[pl.ds(w_base, K)], sem_o.at[0]).wait()
    pltpu.make_async_copy(r.at[1], out.at[pl.ds(w_base, K)], sem_o.at[1]).wait()


@functools.partial(jax.jit, static_argnames=())
def kernel(daytime, weekday, day, daytime_table, weekday_table, day_table):
    dt = daytime.reshape(N).astype(jnp.int32)
    wd = weekday.reshape(N).astype(jnp.int32)
    dy = day.reshape(N).astype(jnp.int32)

    mesh = plsc.VectorSubcoreMesh(core_axis_name="c", subcore_axis_name="s")
    run = pl.kernel(
        _emb_body,
        out_type=jax.ShapeDtypeStruct((N, D), jnp.float32),
        mesh=mesh,
        scratch_types=[
            pltpu.VMEM_SHARED((V_DT, D), jnp.float32),
            pltpu.VMEM_SHARED((V_WD, D), jnp.float32),
            pltpu.VMEM_SHARED((V_DY, D), jnp.float32),
            pltpu.VMEM_SHARED((V_CB, D), jnp.float32),
            pltpu.VMEM((2 * K,), jnp.int32),
            pltpu.VMEM((2 * K,), jnp.int32),
            pltpu.VMEM((2 * K,), jnp.int32),
            pltpu.VMEM((2 * K,), jnp.int32),
            pltpu.VMEM((CB_PER_TILE,), jnp.int32),
            pltpu.VMEM((CB_PER_TILE,), jnp.int32),
            pltpu.VMEM((2, K, D), jnp.float32),
            pltpu.SemaphoreType.DMA((2,)),
            pltpu.SemaphoreType.DMA,
            pltpu.SemaphoreType.DMA((2,)),
        ],
    )
    out = run(dt, wd, dy, daytime_table, weekday_table, day_table)
    return out.reshape(B, L, D)
